# speedup vs baseline: 1.4927x; 1.4927x over previous
import jax
import jax.numpy as jnp
from jax.experimental import pallas as pl


_TS = 512


def _add_kernel(x_ref, p_ref, o_ref):
    o_ref[...] = x_ref[...] + p_ref[...]


def kernel(x, pos_table):
    B, S, D = x.shape
    pos = pos_table[:S]
    grid = (S // _TS, B)
    return pl.pallas_call(
        _add_kernel,
        grid=grid,
        in_specs=[
            pl.BlockSpec((1, _TS, D), lambda i, j: (j, i, 0)),
            pl.BlockSpec((_TS, D), lambda i, j: (i, 0)),
        ],
        out_specs=pl.BlockSpec((1, _TS, D), lambda i, j: (j, i, 0)),
        out_shape=jax.ShapeDtypeStruct((B, S, D), x.dtype),
    )(x, pos)


# speedup vs baseline: 1.6697x; 1.1185x over previous
import jax
import jax.numpy as jnp
from jax.experimental import pallas as pl


_TS = 1024


def _add_kernel(x_ref, p_ref, o_ref):
    o_ref[...] = x_ref[...] + p_ref[...]


def kernel(x, pos_table):
    B, S, D = x.shape
    pos = pos_table[:S]
    grid = (S // _TS, B)
    return pl.pallas_call(
        _add_kernel,
        grid=grid,
        in_specs=[
            pl.BlockSpec((1, _TS, D), lambda i, j: (j, i, 0)),
            pl.BlockSpec((_TS, D), lambda i, j: (i, 0)),
        ],
        out_specs=pl.BlockSpec((1, _TS, D), lambda i, j: (j, i, 0)),
        out_shape=jax.ShapeDtypeStruct((B, S, D), x.dtype),
    )(x, pos)


# speedup vs baseline: 1.7390x; 1.0416x over previous
import jax
import jax.numpy as jnp
from jax.experimental import pallas as pl


_TS = 2048


def _add_kernel(x_ref, p_ref, o_ref):
    o_ref[...] = x_ref[...] + p_ref[...]


def kernel(x, pos_table):
    B, S, D = x.shape
    pos = pos_table[:S]
    grid = (S // _TS, B)
    return pl.pallas_call(
        _add_kernel,
        grid=grid,
        in_specs=[
            pl.BlockSpec((1, _TS, D), lambda i, j: (j, i, 0)),
            pl.BlockSpec((_TS, D), lambda i, j: (i, 0)),
        ],
        out_specs=pl.BlockSpec((1, _TS, D), lambda i, j: (j, i, 0)),
        out_shape=jax.ShapeDtypeStruct((B, S, D), x.dtype),
    )(x, pos)
